# baseline (device time: 61394 ns/iter reference)
import jax
import jax.numpy as jnp
from jax import lax
from jax.experimental import pallas as pl
from jax.experimental.pallas import tpu as pltpu

N_DEV = 16


def kernel(x, Wq, Wo, K_ext, V_ext):
    B, Sq, D = x.shape
    Dq = Wq.shape[1]
    _, Skv, Hq, Dh = K_ext.shape
    C = Sq // N_DEV
    R = Sq // 2
    bf16 = jnp.bfloat16

    def body(x_ref, wq_ref, wo_ref, k_ref, v_ref, out_ref,
             q_s, o_own, l_own, on_chunk, og16, rs_o, rs_l, ag16,
             rso_send, rso_recv, rsl_send, rsl_recv, ag_send, ag_recv):
        my_pos = lax.axis_index("i")

        barrier_sem = pltpu.get_barrier_semaphore()
        for nbr in range(N_DEV):
            @pl.when(nbr != my_pos)
            def _():
                pl.semaphore_signal(
                    barrier_sem, inc=1,
                    device_id=(nbr,), device_id_type=pl.DeviceIdType.MESH,
                )
        pl.semaphore_wait(barrier_sem, N_DEV - 1)

        wq16 = wq_ref[...].astype(bf16)
        for b in range(B):
            q_s[b] = jnp.dot(x_ref[b].astype(bf16), wq16,
                             preferred_element_type=jnp.float32).astype(bf16)

        ones = jnp.ones((Skv, 1), dtype=bf16)
        rs_rdmas = []
        for half in range(2):
            r0 = half * R
            for b in range(B):
                for h in range(Hq):
                    qh = q_s[b, r0:r0 + R, h * Dh:(h + 1) * Dh]
                    kh = k_ref[b, :, h, :].astype(bf16)
                    vh = jnp.concatenate(
                        [v_ref[b, :, h, :].astype(bf16), ones], axis=1)
                    s = lax.dot_general(
                        qh, kh, (((1,), (1,)), ((), ())),
                        preferred_element_type=jnp.float32,
                    ) * 0.125
                    p = jnp.exp(s)
                    o_aug = jnp.dot(p.astype(bf16), vh,
                                    preferred_element_type=jnp.float32)
                    o_own[b, r0:r0 + R, h * Dh:(h + 1) * Dh] = (
                        o_aug[:, :Dh].astype(bf16))
                    c = b * Hq + h
                    l_own[r0:r0 + R, c:c + 1] = o_aug[:, Dh:Dh + 1]

            for dd in range(N_DEV // 2):
                d = half * (N_DEV // 2) + dd
                o_rdma = pltpu.make_async_remote_copy(
                    src_ref=o_own.at[:, pl.ds(d * C, C), :],
                    dst_ref=rs_o.at[my_pos],
                    send_sem=rso_send.at[d], recv_sem=rso_recv.at[my_pos],
                    device_id=(d,), device_id_type=pl.DeviceIdType.MESH,
                )
                l_rdma = pltpu.make_async_remote_copy(
                    src_ref=l_own.at[pl.ds(d * C, C), :],
                    dst_ref=rs_l.at[my_pos],
                    send_sem=rsl_send.at[d], recv_sem=rsl_recv.at[my_pos],
                    device_id=(d,), device_id_type=pl.DeviceIdType.MESH,
                )

                @pl.when(d != my_pos)
                def _():
                    o_rdma.start()
                    l_rdma.start()

                rs_rdmas.append((d, o_rdma, l_rdma))

        rs_o[my_pos] = o_own[:, pl.ds(my_pos * C, C), :]
        rs_l[my_pos] = l_own[pl.ds(my_pos * C, C), :]

        for s in range(N_DEV):
            o_rx = pltpu.make_async_remote_copy(
                src_ref=rs_o.at[s], dst_ref=rs_o.at[s],
                send_sem=rso_send.at[s], recv_sem=rso_recv.at[s],
                device_id=(s,), device_id_type=pl.DeviceIdType.MESH,
            )
            l_rx = pltpu.make_async_remote_copy(
                src_ref=rs_l.at[s], dst_ref=rs_l.at[s],
                send_sem=rsl_send.at[s], recv_sem=rsl_recv.at[s],
                device_id=(s,), device_id_type=pl.DeviceIdType.MESH,
            )

            @pl.when(s != my_pos)
            def _():
                o_rx.wait_recv()
                l_rx.wait_recv()

        on_chunk[...] = rs_o[0].astype(jnp.float32)
        for s in range(1, N_DEV):
            on_chunk[...] += rs_o[s].astype(jnp.float32)
        l_tot = rs_l[0]
        for s in range(1, N_DEV):
            l_tot = l_tot + rs_l[s]

        for b in range(B):
            for h in range(Hq):
                c = b * Hq + h
                on_chunk[b, :, h * Dh:(h + 1) * Dh] = (
                    on_chunk[b, :, h * Dh:(h + 1) * Dh] / l_tot[:, c:c + 1]
                )
        wo16 = wo_ref[...].astype(bf16)
        for b in range(B):
            oc = jnp.dot(on_chunk[b].astype(bf16), wo16,
                         preferred_element_type=jnp.float32)
            out_ref[b, pl.ds(my_pos * C, C), :] = oc
            og16[b] = oc.astype(bf16)

        ag_rdmas = []
        for d in range(N_DEV):
            ag_rdma = pltpu.make_async_remote_copy(
                src_ref=og16,
                dst_ref=ag16.at[my_pos],
                send_sem=ag_send.at[d], recv_sem=ag_recv.at[my_pos],
                device_id=(d,), device_id_type=pl.DeviceIdType.MESH,
            )

            @pl.when(d != my_pos)
            def _():
                ag_rdma.start()

            ag_rdmas.append((d, ag_rdma))

        for s in range(N_DEV):
            ag_rx = pltpu.make_async_remote_copy(
                src_ref=ag16.at[s], dst_ref=ag16.at[s],
                send_sem=ag_send.at[s], recv_sem=ag_recv.at[s],
                device_id=(s,), device_id_type=pl.DeviceIdType.MESH,
            )

            @pl.when(s != my_pos)
            def _():
                ag_rx.wait_recv()
                out_ref[:, pl.ds(s * C, C), :] = ag16[s].astype(jnp.float32)

        for d, o_rdma, l_rdma in rs_rdmas:
            @pl.when(d != my_pos)
            def _():
                o_rdma.wait_send()
                l_rdma.wait_send()
        for d, ag_rdma in ag_rdmas:
            @pl.when(d != my_pos)
            def _():
                ag_rdma.wait_send()

    return pl.pallas_call(
        body,
        out_shape=jax.ShapeDtypeStruct((B, Sq, D), jnp.float32),
        in_specs=[pl.BlockSpec(memory_space=pltpu.VMEM)] * 5,
        out_specs=pl.BlockSpec(memory_space=pltpu.VMEM),
        scratch_shapes=[
            pltpu.VMEM((B, Sq, Dq), bf16),
            pltpu.VMEM((B, Sq, Dq), bf16),
            pltpu.VMEM((Sq, B * Hq), jnp.float32),
            pltpu.VMEM((B, C, Dq), jnp.float32),
            pltpu.VMEM((B, C, D), bf16),
            pltpu.VMEM((N_DEV, B, C, Dq), bf16),
            pltpu.VMEM((N_DEV, C, B * Hq), jnp.float32),
            pltpu.VMEM((N_DEV, B, C, D), bf16),
            pltpu.SemaphoreType.DMA((N_DEV,)),
            pltpu.SemaphoreType.DMA((N_DEV,)),
            pltpu.SemaphoreType.DMA((N_DEV,)),
            pltpu.SemaphoreType.DMA((N_DEV,)),
            pltpu.SemaphoreType.DMA((N_DEV,)),
            pltpu.SemaphoreType.DMA((N_DEV,)),
        ],
        compiler_params=pltpu.CompilerParams(
            collective_id=0, vmem_limit_bytes=110 * 1024 * 1024),
    )(x, Wq, Wo, K_ext, V_ext)


# device time: 54465 ns/iter; 1.1272x vs baseline; 1.1272x over previous
import jax
import jax.numpy as jnp
from jax import lax
from jax.experimental import pallas as pl
from jax.experimental.pallas import tpu as pltpu

N_DEV = 16


def kernel(x, Wq, Wo, K_ext, V_ext):
    B, Sq, D = x.shape
    Dq = Wq.shape[1]
    _, Skv, Hq, Dh = K_ext.shape
    C = Sq // N_DEV
    R = Sq // 2
    bf16 = jnp.bfloat16

    def body(x_ref, wq_ref, wo_ref, k_ref, v_ref, out_ref,
             q_s, o_own, o16, l_own, on_chunk, og16, rs_o, rs_l, ag16,
             rso_send, rso_recv, rsl_send, rsl_recv, ag_send, ag_recv):
        my_pos = lax.axis_index("i")

        barrier_sem = pltpu.get_barrier_semaphore()
        for nbr in range(N_DEV):
            @pl.when(nbr != my_pos)
            def _():
                pl.semaphore_signal(
                    barrier_sem, inc=1,
                    device_id=(nbr,), device_id_type=pl.DeviceIdType.MESH,
                )
        pl.semaphore_wait(barrier_sem, N_DEV - 1)

        for b in range(B):
            q_s[b] = jnp.dot(x_ref[b], wq_ref[...],
                             preferred_element_type=jnp.float32)

        ones = jnp.ones((Skv, 1), dtype=jnp.float32)
        rs_rdmas = []
        for half in range(2):
            r0 = half * R
            for b in range(B):
                for h in range(Hq):
                    qh = q_s[b, r0:r0 + R, h * Dh:(h + 1) * Dh]
                    kh = k_ref[b, :, h, :]
                    vh = v_ref[b, :, h, :]
                    s = lax.dot_general(
                        qh, kh, (((1,), (1,)), ((), ())),
                        preferred_element_type=jnp.float32,
                    ) * 0.125
                    p = jnp.exp(s)
                    o_own[b, r0:r0 + R, h * Dh:(h + 1) * Dh] = jnp.dot(
                        p, vh, preferred_element_type=jnp.float32)
                    c = b * Hq + h
                    l_own[r0:r0 + R, c:c + 1] = jnp.dot(
                        p, ones, preferred_element_type=jnp.float32)

            o16[:, r0:r0 + R, :] = o_own[:, r0:r0 + R, :].astype(bf16)

            for dd in range(N_DEV // 2):
                d = half * (N_DEV // 2) + dd
                o_rdma = pltpu.make_async_remote_copy(
                    src_ref=o16.at[:, pl.ds(d * C, C), :],
                    dst_ref=rs_o.at[my_pos],
                    send_sem=rso_send.at[d], recv_sem=rso_recv.at[my_pos],
                    device_id=(d,), device_id_type=pl.DeviceIdType.MESH,
                )
                l_rdma = pltpu.make_async_remote_copy(
                    src_ref=l_own.at[pl.ds(d * C, C), :],
                    dst_ref=rs_l.at[my_pos],
                    send_sem=rsl_send.at[d], recv_sem=rsl_recv.at[my_pos],
                    device_id=(d,), device_id_type=pl.DeviceIdType.MESH,
                )

                @pl.when(d != my_pos)
                def _():
                    o_rdma.start()
                    l_rdma.start()

                rs_rdmas.append((d, o_rdma, l_rdma))

        rs_o[my_pos] = o16[:, pl.ds(my_pos * C, C), :]
        rs_l[my_pos] = l_own[pl.ds(my_pos * C, C), :]

        for s in range(N_DEV):
            o_rx = pltpu.make_async_remote_copy(
                src_ref=rs_o.at[s], dst_ref=rs_o.at[s],
                send_sem=rso_send.at[s], recv_sem=rso_recv.at[s],
                device_id=(s,), device_id_type=pl.DeviceIdType.MESH,
            )
            l_rx = pltpu.make_async_remote_copy(
                src_ref=rs_l.at[s], dst_ref=rs_l.at[s],
                send_sem=rsl_send.at[s], recv_sem=rsl_recv.at[s],
                device_id=(s,), device_id_type=pl.DeviceIdType.MESH,
            )

            @pl.when(s != my_pos)
            def _():
                o_rx.wait_recv()
                l_rx.wait_recv()

        on_chunk[...] = rs_o[0].astype(jnp.float32)
        for s in range(1, N_DEV):
            on_chunk[...] += rs_o[s].astype(jnp.float32)
        l_tot = rs_l[0]
        for s in range(1, N_DEV):
            l_tot = l_tot + rs_l[s]

        for b in range(B):
            for h in range(Hq):
                c = b * Hq + h
                on_chunk[b, :, h * Dh:(h + 1) * Dh] = (
                    on_chunk[b, :, h * Dh:(h + 1) * Dh] / l_tot[:, c:c + 1]
                )
        for b in range(B):
            oc = jnp.dot(on_chunk[b], wo_ref[...],
                         preferred_element_type=jnp.float32)
            out_ref[b, pl.ds(my_pos * C, C), :] = oc
            og16[b] = oc.astype(bf16)

        ag_rdmas = []
        for d in range(N_DEV):
            ag_rdma = pltpu.make_async_remote_copy(
                src_ref=og16,
                dst_ref=ag16.at[my_pos],
                send_sem=ag_send.at[d], recv_sem=ag_recv.at[my_pos],
                device_id=(d,), device_id_type=pl.DeviceIdType.MESH,
            )

            @pl.when(d != my_pos)
            def _():
                ag_rdma.start()

            ag_rdmas.append((d, ag_rdma))

        for s in range(N_DEV):
            ag_rx = pltpu.make_async_remote_copy(
                src_ref=ag16.at[s], dst_ref=ag16.at[s],
                send_sem=ag_send.at[s], recv_sem=ag_recv.at[s],
                device_id=(s,), device_id_type=pl.DeviceIdType.MESH,
            )

            @pl.when(s != my_pos)
            def _():
                ag_rx.wait_recv()
                out_ref[:, pl.ds(s * C, C), :] = ag16[s].astype(jnp.float32)

        for d, o_rdma, l_rdma in rs_rdmas:
            @pl.when(d != my_pos)
            def _():
                o_rdma.wait_send()
                l_rdma.wait_send()
        for d, ag_rdma in ag_rdmas:
            @pl.when(d != my_pos)
            def _():
                ag_rdma.wait_send()

    return pl.pallas_call(
        body,
        out_shape=jax.ShapeDtypeStruct((B, Sq, D), jnp.float32),
        in_specs=[pl.BlockSpec(memory_space=pltpu.VMEM)] * 5,
        out_specs=pl.BlockSpec(memory_space=pltpu.VMEM),
        scratch_shapes=[
            pltpu.VMEM((B, Sq, Dq), jnp.float32),
            pltpu.VMEM((B, Sq, Dq), jnp.float32),
            pltpu.VMEM((B, Sq, Dq), bf16),
            pltpu.VMEM((Sq, B * Hq), jnp.float32),
            pltpu.VMEM((B, C, Dq), jnp.float32),
            pltpu.VMEM((B, C, D), bf16),
            pltpu.VMEM((N_DEV, B, C, Dq), bf16),
            pltpu.VMEM((N_DEV, C, B * Hq), jnp.float32),
            pltpu.VMEM((N_DEV, B, C, D), bf16),
            pltpu.SemaphoreType.DMA((N_DEV,)),
            pltpu.SemaphoreType.DMA((N_DEV,)),
            pltpu.SemaphoreType.DMA((N_DEV,)),
            pltpu.SemaphoreType.DMA((N_DEV,)),
            pltpu.SemaphoreType.DMA((N_DEV,)),
            pltpu.SemaphoreType.DMA((N_DEV,)),
        ],
        compiler_params=pltpu.CompilerParams(
            collective_id=0, vmem_limit_bytes=110 * 1024 * 1024),
    )(x, Wq, Wo, K_ext, V_ext)


# device time: 47968 ns/iter; 1.2799x vs baseline; 1.1354x over previous
import jax
import jax.numpy as jnp
from jax import lax
from jax.experimental import pallas as pl
from jax.experimental.pallas import tpu as pltpu

N_DEV = 16

DO_RS = True
DO_AG = True
USE_BF16_MXU = True


def kernel(x, Wq, Wo, K_ext, V_ext):
    B, Sq, D = x.shape
    Dq = Wq.shape[1]
    _, Skv, Hq, Dh = K_ext.shape
    C = Sq // N_DEV
    R = Sq // 2
    bf16 = jnp.bfloat16
    cdt = bf16 if USE_BF16_MXU else jnp.float32

    K2 = K_ext.reshape(B, Skv, Hq * Dh).astype(cdt)
    V2 = V_ext.reshape(B, Skv, Hq * Dh).astype(cdt)
    xc = x.astype(cdt)
    wqc = Wq.astype(cdt)

    def body(x_ref, wq_ref, wo_ref, k_ref, v_ref, out_ref,
             q_s, o_own, o16, l_own, on_chunk, og16, rs_o, rs_l, ag16,
             rso_send, rso_recv, rsl_send, rsl_recv, ag_send, ag_recv):
        my_pos = lax.axis_index("i")

        barrier_sem = pltpu.get_barrier_semaphore()
        for nbr in range(N_DEV):
            @pl.when(nbr != my_pos)
            def _():
                pl.semaphore_signal(
                    barrier_sem, inc=1,
                    device_id=(nbr,), device_id_type=pl.DeviceIdType.MESH,
                )
        pl.semaphore_wait(barrier_sem, N_DEV - 1)

        for b in range(B):
            q_s[b] = jnp.dot(x_ref[b], wq_ref[...],
                             preferred_element_type=jnp.float32).astype(cdt)

        rs_rdmas = []
        for half in range(2):
            r0 = half * R
            for b in range(B):
                for h in range(Hq):
                    hs = slice(h * Dh, (h + 1) * Dh)
                    qh = q_s[b, r0:r0 + R, hs]
                    kh = k_ref[b, :, hs]
                    vh = v_ref[b, :, hs]
                    s = lax.dot_general(
                        qh, kh, (((1,), (1,)), ((), ())),
                        preferred_element_type=jnp.float32,
                    ) * 0.125
                    p = jnp.exp(s)
                    c = b * Hq + h
                    l_own[r0:r0 + R, c:c + 1] = jnp.sum(
                        p, axis=1, keepdims=True)
                    o_own[b, r0:r0 + R, hs] = jnp.dot(
                        p.astype(cdt), vh,
                        preferred_element_type=jnp.float32)

            o16[:, r0:r0 + R, :] = o_own[:, r0:r0 + R, :].astype(bf16)

            for dd in range(N_DEV // 2):
                d = half * (N_DEV // 2) + dd
                o_rdma = pltpu.make_async_remote_copy(
                    src_ref=o16.at[:, pl.ds(d * C, C), :],
                    dst_ref=rs_o.at[my_pos],
                    send_sem=rso_send.at[d], recv_sem=rso_recv.at[my_pos],
                    device_id=(d,), device_id_type=pl.DeviceIdType.MESH,
                )
                l_rdma = pltpu.make_async_remote_copy(
                    src_ref=l_own.at[pl.ds(d * C, C), :],
                    dst_ref=rs_l.at[my_pos],
                    send_sem=rsl_send.at[d], recv_sem=rsl_recv.at[my_pos],
                    device_id=(d,), device_id_type=pl.DeviceIdType.MESH,
                )

                if DO_RS:
                    @pl.when(d != my_pos)
                    def _():
                        o_rdma.start()
                        l_rdma.start()

                    rs_rdmas.append((d, o_rdma, l_rdma))

        rs_o[my_pos] = o16[:, pl.ds(my_pos * C, C), :]
        rs_l[my_pos] = l_own[pl.ds(my_pos * C, C), :]

        for s in range(N_DEV):
            o_rx = pltpu.make_async_remote_copy(
                src_ref=rs_o.at[s], dst_ref=rs_o.at[s],
                send_sem=rso_send.at[s], recv_sem=rso_recv.at[s],
                device_id=(s,), device_id_type=pl.DeviceIdType.MESH,
            )
            l_rx = pltpu.make_async_remote_copy(
                src_ref=rs_l.at[s], dst_ref=rs_l.at[s],
                send_sem=rsl_send.at[s], recv_sem=rsl_recv.at[s],
                device_id=(s,), device_id_type=pl.DeviceIdType.MESH,
            )

            if DO_RS:
                @pl.when(s != my_pos)
                def _():
                    o_rx.wait_recv()
                    l_rx.wait_recv()

        on_chunk[...] = rs_o[0].astype(jnp.float32)
        for s in range(1, N_DEV):
            on_chunk[...] += rs_o[s].astype(jnp.float32)
        l_tot = rs_l[0]
        for s in range(1, N_DEV):
            l_tot = l_tot + rs_l[s]

        for b in range(B):
            for h in range(Hq):
                c = b * Hq + h
                on_chunk[b, :, h * Dh:(h + 1) * Dh] = (
                    on_chunk[b, :, h * Dh:(h + 1) * Dh] / l_tot[:, c:c + 1]
                )
        for b in range(B):
            oc = jnp.dot(on_chunk[b], wo_ref[...],
                         preferred_element_type=jnp.float32)
            out_ref[b, pl.ds(my_pos * C, C), :] = oc
            og16[b] = oc.astype(bf16)

        ag_rdmas = []
        for d in range(N_DEV):
            ag_rdma = pltpu.make_async_remote_copy(
                src_ref=og16,
                dst_ref=ag16.at[my_pos],
                send_sem=ag_send.at[d], recv_sem=ag_recv.at[my_pos],
                device_id=(d,), device_id_type=pl.DeviceIdType.MESH,
            )

            if DO_AG:
                @pl.when(d != my_pos)
                def _():
                    ag_rdma.start()

                ag_rdmas.append((d, ag_rdma))

        for s in range(N_DEV):
            ag_rx = pltpu.make_async_remote_copy(
                src_ref=ag16.at[s], dst_ref=ag16.at[s],
                send_sem=ag_send.at[s], recv_sem=ag_recv.at[s],
                device_id=(s,), device_id_type=pl.DeviceIdType.MESH,
            )

            @pl.when(s != my_pos)
            def _():
                if DO_AG:
                    ag_rx.wait_recv()
                out_ref[:, pl.ds(s * C, C), :] = ag16[s].astype(jnp.float32)

        for d, o_rdma, l_rdma in rs_rdmas:
            @pl.when(d != my_pos)
            def _():
                o_rdma.wait_send()
                l_rdma.wait_send()
        for d, ag_rdma in ag_rdmas:
            @pl.when(d != my_pos)
            def _():
                ag_rdma.wait_send()

    return pl.pallas_call(
        body,
        out_shape=jax.ShapeDtypeStruct((B, Sq, D), jnp.float32),
        in_specs=[pl.BlockSpec(memory_space=pltpu.VMEM)] * 5,
        out_specs=pl.BlockSpec(memory_space=pltpu.VMEM),
        scratch_shapes=[
            pltpu.VMEM((B, Sq, Dq), cdt),
            pltpu.VMEM((B, Sq, Dq), jnp.float32),
            pltpu.VMEM((B, Sq, Dq), bf16),
            pltpu.VMEM((Sq, B * Hq), jnp.float32),
            pltpu.VMEM((B, C, Dq), jnp.float32),
            pltpu.VMEM((B, C, D), bf16),
            pltpu.VMEM((N_DEV, B, C, Dq), bf16),
            pltpu.VMEM((N_DEV, C, B * Hq), jnp.float32),
            pltpu.VMEM((N_DEV, B, C, D), bf16),
            pltpu.SemaphoreType.DMA((N_DEV,)),
            pltpu.SemaphoreType.DMA((N_DEV,)),
            pltpu.SemaphoreType.DMA((N_DEV,)),
            pltpu.SemaphoreType.DMA((N_DEV,)),
            pltpu.SemaphoreType.DMA((N_DEV,)),
            pltpu.SemaphoreType.DMA((N_DEV,)),
        ],
        compiler_params=pltpu.CompilerParams(
            collective_id=0, vmem_limit_bytes=110 * 1024 * 1024),
    )(xc, wqc, Wo, K2, V2)
